# baseline (speedup 1.0000x reference)
import jax
import jax.numpy as jnp
from jax.experimental import pallas as pl
from jax.experimental.pallas import tpu as pltpu


def _round_up(a, m):
    return (a + m - 1) // m * m


_TRANS_B = (((1,), (1,)), ((), ()))


def _fused_mlp_kernel(x_ref, w1_ref, b1_ref, w2_ref, b2_ref, o_ref,
                      w1s_ref, w2f_ref, w2s_ref, sem):
    i = pl.program_id(0)

    @pl.when(i == 0)
    def _cast_w1_start_w2():
        pltpu.make_async_copy(w2_ref, w2f_ref, sem).start()
        w1s_ref[...] = w1_ref[...].astype(jnp.bfloat16)

    xb = x_ref[...].astype(jnp.bfloat16)
    h = jax.lax.dot_general(xb, w1s_ref[...], _TRANS_B,
                            preferred_element_type=jnp.float32)
    h = h + b1_ref[...]
    g = 0.5 * h * (1.0 + jax.lax.erf(h * jnp.float32(0.7071067811865476)))
    gb = g.astype(jnp.bfloat16)

    @pl.when(i == 0)
    def _wait_and_cast_w2():
        pltpu.make_async_copy(w2_ref, w2f_ref, sem).wait()
        w2s_ref[...] = w2f_ref[...].astype(jnp.bfloat16)

    o = jax.lax.dot_general(gb, w2s_ref[...], _TRANS_B,
                            preferred_element_type=jnp.float32)
    o_ref[...] = o + b2_ref[...]


def kernel(x, w1, b1, w2, b2, *, tm=1024):
    in_features = x.shape[-1]
    hidden = w1.shape[0]
    out_features = w2.shape[0]
    lead = x.shape[:-1]

    x2 = x.reshape(-1, in_features)
    n_tokens = x2.shape[0]

    tm_eff = max(128, min(_round_up(tm, 128), _round_up(n_tokens, 128)))
    n_pad = _round_up(n_tokens, tm_eff)
    if n_pad != n_tokens:
        x2 = jnp.pad(x2, ((0, n_pad - n_tokens), (0, 0)))
    grid_len = n_pad // tm_eff

    b1r = b1.reshape(1, hidden)
    b2r = b2.reshape(1, out_features)

    flops = 2 * n_pad * (in_features * hidden + hidden * out_features)
    bytes_accessed = 4 * n_pad * (in_features + out_features) + 4 * (
        in_features * hidden + hidden * out_features) + 4 * (hidden + out_features)
    cost = pl.CostEstimate(flops=flops,
                           transcendentals=n_pad * hidden,
                           bytes_accessed=bytes_accessed)

    out = pl.pallas_call(
        _fused_mlp_kernel,
        out_shape=jax.ShapeDtypeStruct((n_pad, out_features), x.dtype),
        grid=(grid_len,),
        in_specs=[
            pl.BlockSpec((tm_eff, in_features), lambda i: (i, 0)),
            pl.BlockSpec((hidden, in_features), lambda i: (0, 0)),
            pl.BlockSpec((1, hidden), lambda i: (0, 0)),
            pl.BlockSpec(memory_space=pl.ANY),
            pl.BlockSpec((1, out_features), lambda i: (0, 0)),
        ],
        out_specs=pl.BlockSpec((tm_eff, out_features), lambda i: (i, 0)),
        scratch_shapes=[
            pltpu.VMEM((hidden, in_features), jnp.bfloat16),
            pltpu.VMEM((out_features, hidden), jnp.float32),
            pltpu.VMEM((out_features, hidden), jnp.bfloat16),
            pltpu.SemaphoreType.DMA,
        ],
        compiler_params=pltpu.CompilerParams(
            dimension_semantics=("arbitrary",),
            vmem_limit_bytes=64 << 20),
        cost_estimate=cost,
    )(x2, w1, b1r, w2, b2r)

    out = out[:n_tokens]
    return out.reshape(*lead, out_features)


# speedup vs baseline: 1.0843x; 1.0843x over previous
import jax
import jax.numpy as jnp
from jax.experimental import pallas as pl
from jax.experimental.pallas import tpu as pltpu


def _round_up(a, m):
    return (a + m - 1) // m * m


_TRANS_B = (((1,), (1,)), ((), ()))


def _fused_mlp_kernel(x_ref, w1_ref, b1_ref, w2_ref, b2_ref, o_ref,
                      w1s_ref, w2s_ref):
    @pl.when(pl.program_id(0) == 0)
    def _cast_weights_once():
        w1s_ref[...] = w1_ref[...].astype(jnp.bfloat16)
        w2s_ref[...] = w2_ref[...].astype(jnp.bfloat16)

    xb = x_ref[...].astype(jnp.bfloat16)
    hidden = w1s_ref.shape[0]
    hc = hidden // 2
    o_acc = None
    for c in range(2):
        h = jax.lax.dot_general(xb, w1s_ref[c * hc:(c + 1) * hc, :], _TRANS_B,
                                preferred_element_type=jnp.float32)
        h = h + b1_ref[:, c * hc:(c + 1) * hc]
        g = 0.5 * h * (1.0 + jax.lax.erf(h * jnp.float32(0.7071067811865476)))
        o_c = jax.lax.dot_general(g.astype(jnp.bfloat16),
                                  w2s_ref[:, c * hc:(c + 1) * hc], _TRANS_B,
                                  preferred_element_type=jnp.float32)
        o_acc = o_c if o_acc is None else o_acc + o_c
    o_ref[...] = o_acc + b2_ref[...]


def kernel(x, w1, b1, w2, b2, *, tm=1024):
    in_features = x.shape[-1]
    hidden = w1.shape[0]
    out_features = w2.shape[0]
    lead = x.shape[:-1]

    x2 = x.reshape(-1, in_features)
    n_tokens = x2.shape[0]

    tm_eff = max(128, min(_round_up(tm, 128), _round_up(n_tokens, 128)))
    n_pad = _round_up(n_tokens, tm_eff)
    if n_pad != n_tokens:
        x2 = jnp.pad(x2, ((0, n_pad - n_tokens), (0, 0)))
    grid_len = n_pad // tm_eff

    b1r = b1.reshape(1, hidden)
    b2r = b2.reshape(1, out_features)

    flops = 2 * n_pad * (in_features * hidden + hidden * out_features)
    bytes_accessed = 4 * n_pad * (in_features + out_features) + 4 * (
        in_features * hidden + hidden * out_features) + 4 * (hidden + out_features)
    cost = pl.CostEstimate(flops=flops,
                           transcendentals=n_pad * hidden,
                           bytes_accessed=bytes_accessed)

    out = pl.pallas_call(
        _fused_mlp_kernel,
        out_shape=jax.ShapeDtypeStruct((n_pad, out_features), x.dtype),
        grid=(grid_len,),
        in_specs=[
            pl.BlockSpec((tm_eff, in_features), lambda i: (i, 0)),
            pl.BlockSpec((hidden, in_features), lambda i: (0, 0)),
            pl.BlockSpec((1, hidden), lambda i: (0, 0)),
            pl.BlockSpec((out_features, hidden), lambda i: (0, 0)),
            pl.BlockSpec((1, out_features), lambda i: (0, 0)),
        ],
        out_specs=pl.BlockSpec((tm_eff, out_features), lambda i: (i, 0)),
        scratch_shapes=[
            pltpu.VMEM((hidden, in_features), jnp.bfloat16),
            pltpu.VMEM((out_features, hidden), jnp.bfloat16),
        ],
        compiler_params=pltpu.CompilerParams(
            dimension_semantics=("arbitrary",),
            vmem_limit_bytes=64 << 20),
        cost_estimate=cost,
    )(x2, w1, b1r, w2, b2r)

    out = out[:n_tokens]
    return out.reshape(*lead, out_features)


# speedup vs baseline: 1.1014x; 1.0158x over previous
import jax
import jax.numpy as jnp
from jax.experimental import pallas as pl
from jax.experimental.pallas import tpu as pltpu


def _round_up(a, m):
    return (a + m - 1) // m * m


_TRANS_B = (((1,), (1,)), ((), ()))


def _fused_mlp_kernel(x_ref, w1_ref, b1_ref, w2_ref, b2_ref, o_ref,
                      w1s_ref, w2s_ref):
    @pl.when(pl.program_id(0) == 0)
    def _cast_weights_once():
        w1s_ref[...] = w1_ref[...].astype(jnp.bfloat16)
        w2s_ref[...] = w2_ref[...].astype(jnp.bfloat16)

    xb = x_ref[...].astype(jnp.bfloat16)
    h = jax.lax.dot_general(xb, w1s_ref[...], _TRANS_B,
                            preferred_element_type=jnp.float32)
    h = h + b1_ref[...]
    g = 0.5 * h * (1.0 + jax.lax.erf(h * jnp.float32(0.7071067811865476)))
    o = jax.lax.dot_general(g.astype(jnp.bfloat16), w2s_ref[...], _TRANS_B,
                            preferred_element_type=jnp.float32)
    o_ref[...] = o + b2_ref[...]


def kernel(x, w1, b1, w2, b2, *, tm=1024):
    in_features = x.shape[-1]
    hidden = w1.shape[0]
    out_features = w2.shape[0]
    lead = x.shape[:-1]

    x2 = x.reshape(-1, in_features)
    n_tokens = x2.shape[0]

    tm_eff = max(128, min(_round_up(tm, 128), _round_up(n_tokens, 128)))
    n_pad = _round_up(n_tokens, tm_eff)
    if n_pad != n_tokens:
        x2 = jnp.pad(x2, ((0, n_pad - n_tokens), (0, 0)))
    grid_len = n_pad // tm_eff

    b1r = b1.reshape(1, hidden)
    b2r = b2.reshape(1, out_features)

    flops = 2 * n_pad * (in_features * hidden + hidden * out_features)
    bytes_accessed = 4 * n_pad * (in_features + out_features) + 4 * (
        in_features * hidden + hidden * out_features) + 4 * (hidden + out_features)
    cost = pl.CostEstimate(flops=flops,
                           transcendentals=n_pad * hidden,
                           bytes_accessed=bytes_accessed)

    out = pl.pallas_call(
        _fused_mlp_kernel,
        out_shape=jax.ShapeDtypeStruct((n_pad, out_features), x.dtype),
        grid=(grid_len,),
        in_specs=[
            pl.BlockSpec((tm_eff, in_features), lambda i: (i, 0)),
            pl.BlockSpec((hidden, in_features), lambda i: (0, 0)),
            pl.BlockSpec((1, hidden), lambda i: (0, 0)),
            pl.BlockSpec((out_features, hidden), lambda i: (0, 0)),
            pl.BlockSpec((1, out_features), lambda i: (0, 0)),
        ],
        out_specs=pl.BlockSpec((tm_eff, out_features), lambda i: (i, 0)),
        scratch_shapes=[
            pltpu.VMEM((hidden, in_features), jnp.bfloat16),
            pltpu.VMEM((out_features, hidden), jnp.bfloat16),
        ],
        compiler_params=pltpu.CompilerParams(
            dimension_semantics=("arbitrary",),
            vmem_limit_bytes=64 << 20),
        cost_estimate=cost,
    )(x2, w1, b1r, w2, b2r)

    out = out[:n_tokens]
    return out.reshape(*lead, out_features)
